# baseline (speedup 1.0000x reference)
import functools

import jax
import jax.numpy as jnp
from jax import lax
from jax.experimental import pallas as pl
from jax.experimental.pallas import tpu as pltpu
from jax.experimental.pallas import tpu_sc as plsc

NTA = 16384
O = 2048
NMAX = 2048
E = 8

B = 256
PAD_N = NTA + E * B
NBLK = PAD_N // B
TN = 512
NT = NMAX // TN

_NC, _NS = 2, 16
_NW = _NC * _NS


def _gather_body(n_rows, chunk, table_hbm, idx_hbm, out_hbm, idx_v,
                 rows0, rows1, sem0, sem1):
    b_per_w = n_rows // _NW
    n_pairs = b_per_w // (2 * chunk)
    wid = lax.axis_index("s") * _NC + lax.axis_index("c")
    base = wid * b_per_w
    pltpu.sync_copy(idx_hbm.at[pl.ds(base, b_per_w)], idx_v)

    def body(k, _):
        i0 = 2 * k * chunk
        i1 = i0 + chunk
        c0 = pltpu.async_copy(table_hbm.at[idx_v.at[pl.ds(i0, chunk)]], rows0, sem0)
        c1 = pltpu.async_copy(table_hbm.at[idx_v.at[pl.ds(i1, chunk)]], rows1, sem1)
        c0.wait()
        pltpu.sync_copy(rows0, out_hbm.at[pl.ds(base + i0, chunk)])
        c1.wait()
        pltpu.sync_copy(rows1, out_hbm.at[pl.ds(base + i1, chunk)])
        return 0

    lax.fori_loop(0, n_pairs, body, 0)


def _sc_row_gather(table, idx, n_rows, chunk=16):
    mesh = plsc.VectorSubcoreMesh(core_axis_name="c", subcore_axis_name="s")
    return pl.kernel(
        functools.partial(_gather_body, n_rows, chunk),
        out_type=jax.ShapeDtypeStruct((n_rows, O), jnp.float32),
        mesh=mesh,
        scratch_types=[
            pltpu.VMEM((n_rows // _NW,), jnp.int32),
            pltpu.VMEM((chunk, O), jnp.float32),
            pltpu.VMEM((chunk, O), jnp.float32),
            pltpu.SemaphoreType.DMA,
            pltpu.SemaphoreType.DMA,
        ],
    )(table, idx)


def _mm_body(expert_ref, x_ref, w_ref, b_ref, o_ref):
    o_ref[...] = jnp.dot(x_ref[...], w_ref[0]) + b_ref[0]


def _expert_matmul(rho_s, W, b, block_expert):
    grid_spec = pltpu.PrefetchScalarGridSpec(
        num_scalar_prefetch=1,
        grid=(NBLK,),
        in_specs=[
            pl.BlockSpec((B, O), lambda i, e_ref: (i, 0)),
            pl.BlockSpec((1, O, NMAX), lambda i, e_ref: (e_ref[i], 0, 0)),
            pl.BlockSpec((1, 1, NMAX), lambda i, e_ref: (e_ref[i], 0, 0)),
        ],
        out_specs=pl.BlockSpec((B, NMAX), lambda i, e_ref: (i, 0)),
    )
    return pl.pallas_call(
        _mm_body,
        grid_spec=grid_spec,
        out_shape=jax.ShapeDtypeStruct((PAD_N, NMAX), jnp.float32),
        compiler_params=pltpu.CompilerParams(
            dimension_semantics=("arbitrary",)),
    )(block_expert, rho_s, W, b.reshape(E, 1, NMAX))


def kernel(rho, symbols, W, b):
    sym = symbols.astype(jnp.int32)

    scramble = (jnp.arange(NTA, dtype=jnp.int32) * 40503) & (NTA - 1)
    sidx = jnp.argsort(sym * NTA + scramble).astype(jnp.int32)
    s_sorted = sym[sidx]
    counts = jnp.bincount(sym, length=E)
    starts = jnp.concatenate([jnp.zeros((1,), counts.dtype), jnp.cumsum(counts)[:-1]])
    padded_counts = ((counts + B - 1) // B) * B
    pcum = jnp.cumsum(padded_counts)
    pstarts = jnp.concatenate([jnp.zeros((1,), pcum.dtype), pcum[:-1]])

    q_of_p = (pstarts[s_sorted] + (jnp.arange(NTA) - starts[s_sorted])).astype(jnp.int32)
    gidx = jnp.zeros((PAD_N,), jnp.int32).at[q_of_p].set(sidx)
    pos = jnp.zeros((NTA,), jnp.int32).at[sidx].set(q_of_p)
    block_expert = jnp.minimum(
        jnp.searchsorted(pcum, jnp.arange(NBLK) * B, side="right"), E - 1
    ).astype(jnp.int32)

    rho_s = _sc_row_gather(rho, gidx, PAD_N)
    y_s = _expert_matmul(rho_s, W, b, block_expert)
    return _sc_row_gather(y_s, pos, NTA)


# speedup vs baseline: 1.5299x; 1.5299x over previous
import functools

import jax
import jax.numpy as jnp
from jax import lax
from jax.experimental import pallas as pl
from jax.experimental.pallas import tpu as pltpu
from jax.experimental.pallas import tpu_sc as plsc

NTA = 16384
O = 2048
NMAX = 2048
E = 8

B = 256
PAD_N = NTA + E * B
NBLK = PAD_N // B
TN = 512
NT = NMAX // TN

_NC, _NS = 2, 16
_NW = _NC * _NS


def _gather_body(n_rows, chunk, table_hbm, idx_hbm, out_hbm, idx_v,
                 rows0, rows1, sem0, sem1):
    b_per_w = n_rows // _NW
    n_pairs = b_per_w // (2 * chunk)
    wid = lax.axis_index("s") * _NC + lax.axis_index("c")
    base = wid * b_per_w
    pltpu.sync_copy(idx_hbm.at[pl.ds(base, b_per_w)], idx_v)

    def body(k, _):
        i0 = 2 * k * chunk
        i1 = i0 + chunk
        c0 = pltpu.async_copy(table_hbm.at[idx_v.at[pl.ds(i0, chunk)]], rows0, sem0)
        c1 = pltpu.async_copy(table_hbm.at[idx_v.at[pl.ds(i1, chunk)]], rows1, sem1)
        c0.wait()
        pltpu.sync_copy(rows0, out_hbm.at[pl.ds(base + i0, chunk)])
        c1.wait()
        pltpu.sync_copy(rows1, out_hbm.at[pl.ds(base + i1, chunk)])
        return 0

    lax.fori_loop(0, n_pairs, body, 0)


def _sc_row_gather(table, idx, n_rows, chunk=16):
    mesh = plsc.VectorSubcoreMesh(core_axis_name="c", subcore_axis_name="s")
    return pl.kernel(
        functools.partial(_gather_body, n_rows, chunk),
        out_type=jax.ShapeDtypeStruct((n_rows, O), jnp.float32),
        mesh=mesh,
        scratch_types=[
            pltpu.VMEM((n_rows // _NW,), jnp.int32),
            pltpu.VMEM((chunk, O), jnp.float32),
            pltpu.VMEM((chunk, O), jnp.float32),
            pltpu.SemaphoreType.DMA,
            pltpu.SemaphoreType.DMA,
        ],
    )(table, idx)


def _mm_body(expert_ref, x_ref, w_ref, b_ref, o_ref):
    o_ref[...] = jnp.dot(x_ref[...], w_ref[0]) + b_ref[0]


def _expert_matmul(rho_s, W, b, block_expert):
    grid_spec = pltpu.PrefetchScalarGridSpec(
        num_scalar_prefetch=1,
        grid=(NBLK,),
        in_specs=[
            pl.BlockSpec((B, O), lambda i, e_ref: (i, 0)),
            pl.BlockSpec((1, O, NMAX), lambda i, e_ref: (e_ref[i], 0, 0)),
            pl.BlockSpec((1, 1, NMAX), lambda i, e_ref: (e_ref[i], 0, 0)),
        ],
        out_specs=pl.BlockSpec((B, NMAX), lambda i, e_ref: (i, 0)),
    )
    return pl.pallas_call(
        _mm_body,
        grid_spec=grid_spec,
        out_shape=jax.ShapeDtypeStruct((PAD_N, NMAX), jnp.float32),
        compiler_params=pltpu.CompilerParams(
            dimension_semantics=("arbitrary",)),
    )(block_expert, rho_s, W, b.reshape(E, 1, NMAX))


def kernel(rho, symbols, W, b):
    sym = symbols.astype(jnp.int32)

    i_arr = jnp.arange(NTA, dtype=jnp.int32)
    scramble = (i_arr * 40503) & (NTA - 1)
    sidx = jnp.argsort(sym * NTA + scramble).astype(jnp.int32)
    inv = jnp.argsort(sidx).astype(jnp.int32)
    e_ids = jnp.arange(E, dtype=jnp.int32)
    counts = (sym[:, None] == e_ids[None, :]).sum(0).astype(jnp.int32)
    starts = jnp.cumsum(counts) - counts
    padded_counts = ((counts + B - 1) // B) * B
    pcum = jnp.cumsum(padded_counts)
    pstarts = pcum - padded_counts

    q_arr = jnp.arange(PAD_N, dtype=jnp.int32)
    e_q = jnp.minimum((q_arr[:, None] >= pcum[None, :]).sum(1), E - 1)
    r_q = q_arr - pstarts[e_q]
    valid = r_q < counts[e_q]
    src_p = jnp.clip(starts[e_q] + r_q, 0, NTA - 1)
    gidx = jnp.where(valid, sidx[src_p], q_arr & (NTA - 1))
    pos = (pstarts[sym] + (inv - starts[sym])).astype(jnp.int32)
    nb = jnp.arange(NBLK, dtype=jnp.int32) * B
    block_expert = jnp.minimum((nb[:, None] >= pcum[None, :]).sum(1), E - 1
                               ).astype(jnp.int32)

    rho_s = _sc_row_gather(rho, gidx, PAD_N)
    y_s = _expert_matmul(rho_s, W, b, block_expert)
    return _sc_row_gather(y_s, pos, NTA)
